# baseline (speedup 1.0000x reference)
import functools

import jax
import jax.numpy as jnp
from jax import lax
from jax.experimental import pallas as pl
from jax.experimental.pallas import tpu as pltpu
from jax.experimental.pallas import tpu_sc as plsc

NUM_CORES = 2
NUM_SUBCORES = 16
NUM_WORKERS = NUM_CORES * NUM_SUBCORES
CHUNK = 800


def kernel(word_ids, table):
    B, L = word_ids.shape
    D = table.shape[1]
    N = B * L
    per_w = N // NUM_WORKERS
    n_chunks = per_w // CHUNK
    assert per_w * NUM_WORKERS == N and n_chunks * CHUNK == per_w
    assert n_chunks % 2 == 0 and n_chunks >= 4

    flat_ids = word_ids.reshape(N)

    mesh = plsc.VectorSubcoreMesh(
        core_axis_name="c",
        subcore_axis_name="s",
        num_cores=NUM_CORES,
        num_subcores=NUM_SUBCORES,
    )

    @functools.partial(
        pl.kernel,
        mesh=mesh,
        out_type=jax.ShapeDtypeStruct((N, D), jnp.float32),
        scratch_types=[
            pltpu.VMEM((CHUNK,), jnp.int32),
            pltpu.VMEM((CHUNK,), jnp.int32),
            pltpu.VMEM((CHUNK, D), jnp.float32),
            pltpu.VMEM((CHUNK, D), jnp.float32),
            pltpu.SemaphoreType.DMA,
            pltpu.SemaphoreType.DMA,
        ],
        compiler_params=pltpu.CompilerParams(use_tc_tiling_on_sc=False),
    )
    def emb_kernel(ids_hbm, table_hbm, out_hbm, idx0, idx1, rows0, rows1,
                   gsem, osem):
        idx = (idx0, idx1)
        rows = (rows0, rows1)
        wid = lax.axis_index("s") * NUM_CORES + lax.axis_index("c")
        base = wid * per_w

        def do_chunk(s, b, drain_prev_out):
            off = base + s * CHUNK
            if drain_prev_out:
                pltpu.make_async_copy(
                    rows[b], out_hbm.at[pl.ds(off, CHUNK)], osem).wait()
            pltpu.sync_copy(ids_hbm.at[pl.ds(off, CHUNK)], idx[b])
            g = pltpu.async_copy(table_hbm.at[idx[b]], rows[b], gsem)
            g.wait()
            pltpu.async_copy(rows[b], out_hbm.at[pl.ds(off, CHUNK)], osem)

        do_chunk(0, 0, False)
        do_chunk(1, 1, False)

        def body(i, carry):
            s = 2 + 2 * i
            do_chunk(s, 0, True)
            do_chunk(s + 1, 1, True)
            return carry

        lax.fori_loop(0, (n_chunks - 2) // 2, body, 0)

        for b in (0, 1):
            pltpu.make_async_copy(
                rows[b], out_hbm.at[pl.ds(base, CHUNK)], osem).wait()

    out = emb_kernel(flat_ids, table)
    return out.reshape(B, L, D)
